# baseline (speedup 1.0000x reference)
import functools

import jax
import jax.numpy as jnp
from jax import lax
from jax.experimental import pallas as pl
from jax.experimental.pallas import tpu as pltpu
from jax.experimental.pallas import tpu_sc as plsc

N_NODES = 10000
N_EDGES = 320000
D_FEAT = 128
D_EDGE = 16
UNITS = 128
EPS = 1e-3

NC = 2
NS = 16
NW = NC * NS
CHUNK = 40
N_CHUNKS = 252
EDGES_PER_WORKER = N_CHUNKS * CHUNK
N_EDGES_PAD = NW * EDGES_PER_WORKER
N_PAD = 10240
ROWS_PER_TILE = N_PAD // NS


def _proj_nodes_body(nf_ref, w_ref, out_ref):
    out_ref[...] = jnp.dot(nf_ref[...], w_ref[...],
                           preferred_element_type=jnp.float32)


def _proj_edges_body(ef_ref, w_ref, b_ref, out_ref):
    out_ref[...] = jnp.dot(ef_ref[...], w_ref[...],
                           preferred_element_type=jnp.float32) + b_ref[...]


def _project(node_features, edge_features, w_node, w_edge, b2d):
    p = pl.pallas_call(
        _proj_nodes_body,
        grid=(5,),
        in_specs=[
            pl.BlockSpec((2000, D_FEAT), lambda i: (i, 0)),
            pl.BlockSpec((D_FEAT, UNITS), lambda i: (0, 0)),
        ],
        out_specs=pl.BlockSpec((2000, UNITS), lambda i: (i, 0)),
        out_shape=jax.ShapeDtypeStruct((N_NODES, UNITS), jnp.float32),
    )(node_features, w_node)

    e = pl.pallas_call(
        _proj_edges_body,
        grid=(40,),
        in_specs=[
            pl.BlockSpec((N_EDGES_PAD // 40, D_EDGE), lambda i: (i, 0)),
            pl.BlockSpec((D_EDGE, UNITS), lambda i: (0, 0)),
            pl.BlockSpec((1, UNITS), lambda i: (0, 0)),
        ],
        out_specs=pl.BlockSpec((N_EDGES_PAD // 40, UNITS), lambda i: (i, 0)),
        out_shape=jax.ShapeDtypeStruct((N_EDGES_PAD, UNITS), jnp.float32),
    )(edge_features, w_edge, b2d)
    return p, e


def _silu_chunk(g_v, e_v):
    def _row(r, _):
        for j in range(8):
            x = g_v[r, pl.ds(j * 16, 16)] + e_v[r, pl.ds(j * 16, 16)]
            e_v[r, pl.ds(j * 16, 16)] = x / (1.0 + jnp.exp(-x))
        return 0
    lax.fori_loop(0, CHUNK, _row, 0)


def _sc_body(p_hbm, e_hbm, idx3_hbm, part_hbm, acc,
             i0, i1, i2, i3, i4, i5, g0, g1, g2, e0, e1, e2,
             gsem0, gsem1, gsem2, esem0, esem1, esem2,
             ssem0, ssem1, ssem2, isem0, isem1, isem2):
    cid = lax.axis_index("c")
    sid = lax.axis_index("s")
    wid = sid * NC + cid
    ibuf = (i0, i1, i2, i3, i4, i5)
    gbuf = (g0, g1, g2)
    ebuf = (e0, e1, e2)
    gsem = (gsem0, gsem1, gsem2)
    esem = (esem0, esem1, esem2)
    ssem = (ssem0, ssem1, ssem2)
    isem = (isem0, isem1, isem2)

    def _zrow(r, _):
        for j in range(8):
            g0[r, pl.ds(j * 16, 16)] = jnp.zeros((16,), jnp.float32)
        return 0
    lax.fori_loop(0, CHUNK, _zrow, 0)
    for k in range(ROWS_PER_TILE // CHUNK):
        pltpu.sync_copy(g0, acc.at[pl.ds(sid * ROWS_PER_TILE + k * CHUNK,
                                         CHUNK)])
    plsc.subcore_barrier()

    base = wid * EDGES_PER_WORKER
    ibase = wid * N_CHUNKS

    def _scatter_drain(c, s, b):
        pltpu.make_async_copy(ebuf[b], acc.at[ibuf[s].at[1]], ssem[b]).wait()

    def _launch(c, s, b, first=False):
        if not first:
            @pl.when(c >= 3)
            def _():
                _scatter_drain(c - 3, (s + 3) % 6, b)
        pltpu.make_async_copy(idx3_hbm.at[ibase + c], ibuf[s], isem[b]).wait()
        pltpu.async_copy(p_hbm.at[ibuf[s].at[0]], gbuf[b], gsem[b])
        pltpu.async_copy(e_hbm.at[pl.ds(base + c * CHUNK, CHUNK)],
                         ebuf[b], esem[b])

        @pl.when(c + 3 < N_CHUNKS)
        def _():
            pltpu.async_copy(idx3_hbm.at[ibase + c + 3], ibuf[(s + 3) % 6],
                             isem[b])

    def _finish(c, s, b):
        pltpu.make_async_copy(p_hbm.at[ibuf[s].at[0]], gbuf[b],
                              gsem[b]).wait()
        pltpu.make_async_copy(e_hbm.at[pl.ds(base + c * CHUNK, CHUNK)],
                              ebuf[b], esem[b]).wait()
        _silu_chunk(gbuf[b], ebuf[b])
        pltpu.async_copy(ebuf[b], acc.at[ibuf[s].at[1]], ssem[b], add=True)

        @pl.when(c + 2 < N_CHUNKS)
        def _():
            _launch(c + 2, (s + 2) % 6, (b + 2) % 3)

    for k in range(3):
        pltpu.async_copy(idx3_hbm.at[ibase + k], ibuf[k], isem[k])
    _launch(0, 0, 0, first=True)
    _launch(1, 1, 1, first=True)

    def _six(i, _):
        c = 6 * i
        for k in range(6):
            _finish(c + k, k, k % 3)
        return 0
    lax.fori_loop(0, N_CHUNKS // 6, _six, 0)

    for c in (N_CHUNKS - 3, N_CHUNKS - 2, N_CHUNKS - 1):
        _scatter_drain(c, c % 6, c % 3)

    plsc.subcore_barrier()
    for k in range(ROWS_PER_TILE // CHUNK):
        r0 = sid * ROWS_PER_TILE + k * CHUNK
        pltpu.sync_copy(acc.at[pl.ds(r0, CHUNK)],
                        part_hbm.at[cid, pl.ds(r0, CHUNK)])


def _sc_aggregate(p, e, idx3):
    mesh = plsc.VectorSubcoreMesh(core_axis_name="c", subcore_axis_name="s")
    f = pl.kernel(
        _sc_body,
        out_type=jax.ShapeDtypeStruct((NC, N_PAD, UNITS), jnp.float32),
        mesh=mesh,
        scratch_types=(
            [pltpu.VMEM_SHARED((N_PAD, UNITS), jnp.float32)]
            + [pltpu.VMEM((2, CHUNK), jnp.int32)] * 6
            + [pltpu.VMEM((CHUNK, UNITS), jnp.float32)] * 3
            + [pltpu.VMEM((CHUNK, UNITS), jnp.float32)] * 3
            + [pltpu.SemaphoreType.DMA] * 12
        ),
    )
    return f(p, e, idx3)


def _ln_body(part_ref, g_ref, b_ref, out_ref):
    s = part_ref[0] + part_ref[1]
    mean = jnp.mean(s, axis=-1, keepdims=True)
    var = jnp.mean(jnp.square(s - mean), axis=-1, keepdims=True)
    out_ref[...] = (s - mean) * lax.rsqrt(var + EPS) * g_ref[...] + b_ref[...]


def _layernorm(partials, gamma2d, beta2d):
    return pl.pallas_call(
        _ln_body,
        grid=(5,),
        in_specs=[
            pl.BlockSpec((NC, 2000, UNITS), lambda i: (0, i, 0)),
            pl.BlockSpec((1, UNITS), lambda i: (0, 0)),
            pl.BlockSpec((1, UNITS), lambda i: (0, 0)),
        ],
        out_specs=pl.BlockSpec((2000, UNITS), lambda i: (i, 0)),
        out_shape=jax.ShapeDtypeStruct((N_NODES, UNITS), jnp.float32),
    )(partials, gamma2d, beta2d)


@jax.jit
def kernel(node_features, edge_index, edge_features, W, b, gamma, beta):
    n_extra = N_EDGES_PAD - N_EDGES
    idx = edge_index.astype(jnp.int32)
    pad_vals = jnp.stack([jnp.zeros((n_extra,), jnp.int32),
                          jnp.full((n_extra,), N_NODES, jnp.int32)])
    idx3 = (jnp.concatenate([idx, pad_vals], axis=1)
            .reshape(2, NW, N_CHUNKS, CHUNK)
            .transpose(1, 2, 0, 3)
            .reshape(NW * N_CHUNKS, 2, CHUNK))
    ef_pad = jnp.concatenate(
        [edge_features, jnp.zeros((n_extra, D_EDGE), jnp.float32)])
    w_node = W[:D_FEAT]
    w_edge = W[D_FEAT:]
    p, e = _project(node_features, ef_pad, w_node, w_edge,
                    b.reshape(1, UNITS))
    partials = _sc_aggregate(p, e, idx3)
    return _layernorm(partials, gamma.reshape(1, UNITS),
                      beta.reshape(1, UNITS))
